# baseline (speedup 1.0000x reference)
import functools

import jax
import jax.numpy as jnp
from jax import lax
from jax.experimental import pallas as pl
from jax.experimental.pallas import tpu as pltpu
from jax.experimental.pallas import tpu_sc as plsc

_Q = 0.7
_K = 0.5
_C = (1.0 - _K**_Q) / _Q

_BATCH = 16384
_NCLS = 1000
_CPAD = 1024
_BR = 512
_NS = 4


def _gather_w_sc(weight_flat, indexes):
    info = plsc.get_sparse_core_info()
    nc, ns = info.num_cores, info.num_subcores
    nw = nc * ns
    b = indexes.shape[0]
    b_per_w = b // nw
    mesh = plsc.VectorSubcoreMesh(core_axis_name="c", subcore_axis_name="s")

    @functools.partial(
        pl.kernel,
        mesh=mesh,
        out_type=jax.ShapeDtypeStruct((b,), jnp.float32),
        scratch_types=[
            pltpu.VMEM((b_per_w,), jnp.int32),
            pltpu.VMEM((b_per_w,), jnp.float32),
            pltpu.SemaphoreType.DMA,
        ],
    )
    def gather_kernel(table_hbm, idx_hbm, out_hbm, idx_v, rows_v, sem):
        wid = lax.axis_index("s") * nc + lax.axis_index("c")
        base = wid * b_per_w
        pltpu.sync_copy(idx_hbm.at[pl.ds(base, b_per_w)], idx_v)
        pltpu.async_copy(table_hbm.at[idx_v], rows_v, sem).wait()
        pltpu.sync_copy(rows_v, out_hbm.at[pl.ds(base, b_per_w)])

    return gather_kernel(weight_flat, indexes)


def _block_loss(x, t, w):
    col = lax.broadcasted_iota(jnp.int32, x.shape, 1)
    valid = col < _NCLS
    t = jnp.where(valid, t, -1.0)
    x = jnp.where(valid, x, -3e38)
    tmax = jnp.max(t, axis=1, keepdims=True)
    jstar = jnp.min(jnp.where(t == tmax, col, _CPAD), axis=1, keepdims=True)
    g = jnp.sum(jnp.where(col == jstar, x, 0.0), axis=1)
    m = jnp.max(x, axis=1)
    s = jnp.sum(jnp.exp(x - m[:, None]), axis=1)
    yg = jnp.exp(g - m) / s
    a = (1.0 - yg**_Q) / _Q - _C
    return jnp.sum(a * w)


def _dense_body(*refs):
    out_ref = refs[-1]
    xs = refs[:_NS]
    ts = refs[_NS : 2 * _NS]
    ws = refs[2 * _NS : 3 * _NS]
    partial = 0.0
    for k in range(_NS):
        partial += _block_loss(xs[k][...], ts[k][...], ws[k][0, 0, :])
    partial *= 1.0 / _BATCH

    @pl.when(pl.program_id(0) == 0)
    def _():
        out_ref[0, 0] = 0.0

    out_ref[0, 0] += partial


def _dense_loss_tc(logits, targets, w):
    nb = _BATCH // _BR
    steps = nb // _NS
    w3 = w.reshape(nb, 1, _BR)
    dense_specs = [
        pl.BlockSpec((_BR, _CPAD), (lambda i, k=k: (k * steps + i, 0)))
        for k in range(_NS)
    ]
    w_specs = [
        pl.BlockSpec((1, 1, _BR), (lambda i, k=k: (k * steps + i, 0, 0)))
        for k in range(_NS)
    ]
    out = pl.pallas_call(
        _dense_body,
        grid=(steps,),
        in_specs=dense_specs + dense_specs + w_specs,
        out_specs=pl.BlockSpec(
            (1, 1), lambda i: (0, 0), memory_space=pltpu.SMEM
        ),
        out_shape=jax.ShapeDtypeStruct((1, 1), jnp.float32),
    )(*([logits] * _NS + [targets] * _NS + [w3] * _NS))
    return out[0, 0]


def kernel(logits, targets, indexes, weight):
    w = _gather_w_sc(weight.reshape(-1), indexes)
    return _dense_loss_tc(logits, targets, w)


# speedup vs baseline: 2.0841x; 2.0841x over previous
import functools

import jax
import jax.numpy as jnp
from jax import lax
from jax.experimental import pallas as pl
from jax.experimental.pallas import tpu as pltpu
from jax.experimental.pallas import tpu_sc as plsc

_Q = 0.7
_K = 0.5
_C = (1.0 - _K**_Q) / _Q

_BATCH = 16384
_NCLS = 1000
_CPAD = 1024
_BR = 512
_NS = 4


def _gather_w_sc(weight_flat, indexes):
    info = plsc.get_sparse_core_info()
    nc, ns = info.num_cores, info.num_subcores
    nw = nc * ns
    b = indexes.shape[0]
    b_per_w = b // nw
    mesh = plsc.VectorSubcoreMesh(core_axis_name="c", subcore_axis_name="s")

    @functools.partial(
        pl.kernel,
        mesh=mesh,
        out_type=jax.ShapeDtypeStruct((b,), jnp.float32),
        scratch_types=[
            pltpu.VMEM((b_per_w,), jnp.int32),
            pltpu.VMEM((b_per_w,), jnp.float32),
            pltpu.SemaphoreType.DMA,
        ],
    )
    def gather_kernel(table_hbm, idx_hbm, out_hbm, idx_v, rows_v, sem):
        wid = lax.axis_index("s") * nc + lax.axis_index("c")
        base = wid * b_per_w
        pltpu.sync_copy(idx_hbm.at[pl.ds(base, b_per_w)], idx_v)
        pltpu.async_copy(table_hbm.at[idx_v], rows_v, sem).wait()
        pltpu.sync_copy(rows_v, out_hbm.at[pl.ds(base, b_per_w)])

    return gather_kernel(weight_flat, indexes)


def _block_loss(x, t, w):
    row = lax.broadcasted_iota(jnp.int32, x.shape, 0)
    tmax = jnp.max(t, axis=0, keepdims=True)
    jstar = jnp.min(jnp.where(t == tmax, row, _NCLS), axis=0, keepdims=True)
    g = jnp.sum(jnp.where(row == jstar, x, 0.0), axis=0)
    m = jnp.max(x, axis=0)
    s = jnp.sum(jnp.exp(x - m[None, :]), axis=0)
    yg = jnp.exp(g - m) / s
    a = (1.0 - yg**_Q) / _Q - _C
    return jnp.sum(a * w)


def _dense_body(*refs):
    out_ref = refs[-1]
    xs = refs[:_NS]
    ts = refs[_NS : 2 * _NS]
    ws = refs[2 * _NS : 3 * _NS]
    partial = 0.0
    for k in range(_NS):
        partial += _block_loss(xs[k][...], ts[k][...], ws[k][0, 0, :])
    partial *= 1.0 / _BATCH

    @pl.when(pl.program_id(0) == 0)
    def _():
        out_ref[0, 0] = 0.0

    out_ref[0, 0] += partial


def _dense_loss_tc(logits, targets, w):
    nb = _BATCH // _BR
    steps = nb // _NS
    xt = logits.T
    tt = targets.T
    w3 = w.reshape(nb, 1, _BR)
    dense_specs = [
        pl.BlockSpec((_NCLS, _BR), (lambda i, k=k: (0, k * steps + i)))
        for k in range(_NS)
    ]
    w_specs = [
        pl.BlockSpec((1, 1, _BR), (lambda i, k=k: (k * steps + i, 0, 0)))
        for k in range(_NS)
    ]
    out = pl.pallas_call(
        _dense_body,
        grid=(steps,),
        in_specs=dense_specs + dense_specs + w_specs,
        out_specs=pl.BlockSpec(
            (1, 1), lambda i: (0, 0), memory_space=pltpu.SMEM
        ),
        out_shape=jax.ShapeDtypeStruct((1, 1), jnp.float32),
    )(*([xt] * _NS + [tt] * _NS + [w3] * _NS))
    return out[0, 0]


def kernel(logits, targets, indexes, weight):
    w = _gather_w_sc(weight.reshape(-1), indexes)
    return _dense_loss_tc(logits, targets, w)


# speedup vs baseline: 2.1041x; 1.0096x over previous
import functools

import jax
import jax.numpy as jnp
from jax import lax
from jax.experimental import pallas as pl
from jax.experimental.pallas import tpu as pltpu
from jax.experimental.pallas import tpu_sc as plsc

_Q = 0.7
_K = 0.5
_C = (1.0 - _K**_Q) / _Q

_BATCH = 16384
_NCLS = 1000
_CPAD = 1024
_BR = 512
_NS = 4


def _gather_w_sc(weight, indexes):
    info = plsc.get_sparse_core_info()
    nc, ns = info.num_cores, info.num_subcores
    nw = nc * ns
    b = indexes.shape[0]
    b_per_w = b // nw
    mesh = plsc.VectorSubcoreMesh(core_axis_name="c", subcore_axis_name="s")

    @functools.partial(
        pl.kernel,
        mesh=mesh,
        out_type=jax.ShapeDtypeStruct((b,), jnp.float32),
        scratch_types=[
            pltpu.VMEM((b_per_w,), jnp.int32),
            pltpu.VMEM((b_per_w,), jnp.float32),
            pltpu.SemaphoreType.DMA,
        ],
    )
    def gather_kernel(table_hbm, idx_hbm, out_hbm, idx_v, rows_v, sem):
        wid = lax.axis_index("s") * nc + lax.axis_index("c")
        base = wid * b_per_w
        pltpu.sync_copy(idx_hbm.at[pl.ds(base, b_per_w)], idx_v)
        pltpu.async_copy(table_hbm.at[idx_v], rows_v, sem).wait()
        pltpu.sync_copy(rows_v, out_hbm.at[pl.ds(base, b_per_w)])

    return gather_kernel(weight.reshape(-1), indexes)


def _block_loss(x, t, w):
    row = lax.broadcasted_iota(jnp.int32, x.shape, 0)
    tmax = jnp.max(t, axis=0, keepdims=True)
    jstar = jnp.min(jnp.where(t == tmax, row, _NCLS), axis=0, keepdims=True)
    g = jnp.sum(jnp.where(row == jstar, x, 0.0), axis=0)
    m = jnp.max(x, axis=0)
    s = jnp.sum(jnp.exp(x - m[None, :]), axis=0)
    yg = jnp.exp(g - m) / s
    a = (1.0 - yg**_Q) / _Q - _C
    return jnp.sum(a * w)


def _dense_body(*refs):
    out_ref = refs[-1]
    xs = refs[:_NS]
    ts = refs[_NS : 2 * _NS]
    ws = refs[2 * _NS : 3 * _NS]
    partial = 0.0
    for k in range(_NS):
        partial += _block_loss(xs[k][...], ts[k][...], ws[k][0, 0, :])
    partial *= 1.0 / _BATCH

    @pl.when(pl.program_id(0) == 0)
    def _():
        out_ref[0, 0] = 0.0

    out_ref[0, 0] += partial


def _dense_loss_tc(logits, targets, w):
    nb = _BATCH // _BR
    steps = nb // _NS
    xt = logits.T
    tt = targets.T
    w3 = w.reshape(nb, 1, _BR)
    dense_specs = [
        pl.BlockSpec((_NCLS, _BR), (lambda i, k=k: (0, k * steps + i)))
        for k in range(_NS)
    ]
    w_specs = [
        pl.BlockSpec((1, 1, _BR), (lambda i, k=k: (k * steps + i, 0, 0)))
        for k in range(_NS)
    ]
    out = pl.pallas_call(
        _dense_body,
        grid=(steps,),
        in_specs=dense_specs + dense_specs + w_specs,
        out_specs=pl.BlockSpec(
            (1, 1), lambda i: (0, 0), memory_space=pltpu.SMEM
        ),
        out_shape=jax.ShapeDtypeStruct((1, 1), jnp.float32),
    )(*([xt] * _NS + [tt] * _NS + [w3] * _NS))
    return out[0, 0]


def kernel(logits, targets, indexes, weight):
    w = _gather_w_sc(weight, indexes)
    return _dense_loss_tc(logits, targets, w)
